# baseline (speedup 1.0000x reference)
import functools
import math

import jax
import jax.numpy as jnp
from jax import lax
from jax.experimental import pallas as pl
from jax.experimental.pallas import tpu as pltpu
from jax.experimental.pallas import tpu_sc as plsc

EMB = 64
SCALE = float(math.sqrt(EMB))
NC, NS, L = 2, 16, 16
NW = NC * NS
G = 128


def kernel(tokens, table):
    B = tokens.shape[0] * tokens.shape[1]
    V = table.shape[0]
    b_per_w = B // NW
    W = 512
    steps = b_per_w // W
    assert b_per_w % W == 0

    idx = tokens.reshape(B).astype(jnp.int32)
    table_w = table.reshape(V // 2, 2 * EMB)
    mesh = plsc.VectorSubcoreMesh(core_axis_name="c", subcore_axis_name="s")

    @functools.partial(
        pl.kernel,
        mesh=mesh,
        out_type=jax.ShapeDtypeStruct((B // 2, 2 * EMB), jnp.float32),
        scratch_types=[
            pltpu.VMEM((W,), jnp.int32),
            pltpu.VMEM((W // G, G), jnp.int32),
            pltpu.VMEM((W, 2 * EMB), jnp.float32),
            pltpu.VMEM((W // 2, 2 * EMB), jnp.float32),
            pltpu.SemaphoreType.DMA,
        ],
        compiler_params=pltpu.CompilerParams(needs_layout_passes=False),
    )
    def emb_kernel(idx_hbm, table_hbm, out_hbm, idx_v, widx_v, gath_v, out_v, sem):
        wid = lax.axis_index("s") * NC + lax.axis_index("c")
        base = wid * b_per_w
        ivec = lax.iota(jnp.int32, 16)

        @pl.loop(0, steps)
        def _(i):
            off = pl.multiple_of(base + i * W, W)
            pltpu.sync_copy(idx_hbm.at[pl.ds(off, W)], idx_v)
            for j in range(W // G):
                for m in range(G // 16):
                    widx_v[j, pl.ds(m * 16, 16)] = (
                        idx_v[pl.ds(j * G + m * 16, 16)] >> 1
                    )
            for j in range(W // G):
                pltpu.async_copy(
                    table_hbm.at[widx_v.at[j]],
                    gath_v.at[pl.ds(j * G, G)],
                    sem,
                )
            for j in range(W // G):
                pltpu.make_async_copy(
                    table_hbm.at[widx_v.at[j]],
                    gath_v.at[pl.ds(j * G, G)],
                    sem,
                ).wait()

            @pl.loop(0, W // 16)
            def _(g):
                r0 = g * 16
                rvec = jnp.full((16,), r0, jnp.int32) + ivec
                idxvec = idx_v[pl.ds(r0, 16)]
                gbase = (idxvec & 1) << 6
                o_rowv = rvec >> 1
                o_base = (rvec & 1) << 6
                for cc in range(2 * EMB // 2):
                    vals = plsc.load_gather(gath_v, [rvec, gbase + cc])
                    plsc.store_scatter(
                        out_v, [o_rowv, o_base + cc], vals * SCALE
                    )

            pltpu.sync_copy(out_v, out_hbm.at[pl.ds(pl.multiple_of(off // 2, W // 2), W // 2)])

    out = emb_kernel(idx, table_w)
    return out.reshape(tokens.shape + (EMB,))


# speedup vs baseline: 1.9671x; 1.9671x over previous
import functools
import math

import jax
import jax.numpy as jnp
from jax import lax
from jax.experimental import pallas as pl
from jax.experimental.pallas import tpu as pltpu
from jax.experimental.pallas import tpu_sc as plsc

EMB = 64
SCALE = float(math.sqrt(EMB))
NC, NS = 2, 16
NW = NC * NS
G = 128
TBLK = 512


def _transpose_scale_pad(tT):
    V = tT.shape[1]

    def body(x_ref, o_ref):
        y = jnp.swapaxes(x_ref[...], 0, 1) * SCALE
        o_ref[...] = jnp.concatenate([y, jnp.zeros_like(y)], axis=1)

    return pl.pallas_call(
        body,
        grid=(pl.cdiv(V, TBLK),),
        in_specs=[pl.BlockSpec((EMB, TBLK), lambda i: (0, i))],
        out_specs=pl.BlockSpec((TBLK, 2 * EMB), lambda i: (i, 0)),
        out_shape=jax.ShapeDtypeStruct((V, 2 * EMB), jnp.float32),
    )(tT)


def kernel(tokens, table):
    B = tokens.shape[0] * tokens.shape[1]
    V = table.shape[0]
    b_per_w = B // NW
    MACRO = 1024
    HALF = 256
    macros = b_per_w // MACRO
    assert b_per_w % MACRO == 0

    table2 = _transpose_scale_pad(jnp.swapaxes(table, 0, 1))
    idx = tokens.reshape(B // G, G).astype(jnp.int32)
    mesh = plsc.VectorSubcoreMesh(core_axis_name="c", subcore_axis_name="s")

    @functools.partial(
        pl.kernel,
        mesh=mesh,
        out_type=jax.ShapeDtypeStruct((B, 2 * EMB), jnp.float32),
        scratch_types=[
            pltpu.VMEM((MACRO // G, G), jnp.int32),
            pltpu.VMEM((HALF, 2 * EMB), jnp.float32),
            pltpu.VMEM((HALF, 2 * EMB), jnp.float32),
            pltpu.SemaphoreType.DMA,
            pltpu.SemaphoreType.DMA,
        ],
    )
    def emb_kernel(idx_hbm, table_hbm, out_hbm, idx_v, gath0, gath1, semA, semB):
        wid = lax.axis_index("s") * NC + lax.axis_index("c")
        base = wid * b_per_w

        @pl.loop(0, macros)
        def _(i):
            off = pl.multiple_of(base + i * MACRO, MACRO)
            row0 = pl.multiple_of(off // G, MACRO // G)
            pltpu.sync_copy(idx_hbm.at[pl.ds(row0, MACRO // G)], idx_v)
            gaths = (gath0, gath1)
            sems = (semA, semB)
            nq = MACRO // HALF

            def fire(q):
                for j in range(HALF // G):
                    pltpu.async_copy(
                        table_hbm.at[idx_v.at[q * (HALF // G) + j]],
                        gaths[q % 2].at[pl.ds(j * G, G)],
                        sems[q % 2],
                    )

            def drain_out(q):
                for j in range(HALF // G):
                    pltpu.make_async_copy(
                        table_hbm.at[idx_v.at[q * (HALF // G) + j]],
                        gaths[q % 2].at[pl.ds(j * G, G)],
                        sems[q % 2],
                    ).wait()
                pltpu.sync_copy(
                    gaths[q % 2],
                    out_hbm.at[pl.ds(off + q * HALF, HALF)],
                )

            fire(0)
            for q in range(1, nq):
                fire(q)
                drain_out(q - 1)
            drain_out(nq - 1)

    out = emb_kernel(idx, table2)
    return out[:, :EMB].reshape(tokens.shape + (EMB,))


# speedup vs baseline: 2.4369x; 1.2388x over previous
import functools
import math

import jax
import jax.numpy as jnp
from jax import lax
from jax.experimental import pallas as pl
from jax.experimental.pallas import tpu as pltpu
from jax.experimental.pallas import tpu_sc as plsc

EMB = 64
SCALE = float(math.sqrt(EMB))
NC, NS = 2, 16
NW = NC * NS
G = 128
TBLK = 1024


def _transpose_scale_pad(tT):
    V = tT.shape[1]
    P = jnp.concatenate(
        [jnp.eye(EMB, dtype=jnp.float32) * SCALE,
         jnp.zeros((EMB, EMB), jnp.float32)], axis=1)

    def body(x_ref, p_ref, o_ref):
        o_ref[...] = jax.lax.dot_general(
            x_ref[...], p_ref[...], (((0,), (0,)), ((), ())),
            precision=jax.lax.Precision.HIGHEST)

    return pl.pallas_call(
        body,
        grid=(pl.cdiv(V, TBLK),),
        in_specs=[pl.BlockSpec((EMB, TBLK), lambda i: (0, i)),
                  pl.BlockSpec((EMB, 2 * EMB), lambda i: (0, 0))],
        out_specs=pl.BlockSpec((TBLK, 2 * EMB), lambda i: (i, 0)),
        out_shape=jax.ShapeDtypeStruct((V, 2 * EMB), jnp.float32),
        compiler_params=pltpu.CompilerParams(
            dimension_semantics=("parallel",)),
    )(tT, P)


def kernel(tokens, table):
    B = tokens.shape[0] * tokens.shape[1]
    V = table.shape[0]
    b_per_w = B // NW
    MACRO = 1024
    HALF = 256
    macros = b_per_w // MACRO
    assert b_per_w % MACRO == 0

    table2 = _transpose_scale_pad(jnp.swapaxes(table, 0, 1))
    idx = tokens.reshape(B // G, G).astype(jnp.int32)
    mesh = plsc.VectorSubcoreMesh(core_axis_name="c", subcore_axis_name="s")

    @functools.partial(
        pl.kernel,
        mesh=mesh,
        out_type=jax.ShapeDtypeStruct((B, 2 * EMB), jnp.float32),
        scratch_types=[
            pltpu.VMEM((MACRO // G, G), jnp.int32),
            pltpu.VMEM((HALF, 2 * EMB), jnp.float32),
            pltpu.VMEM((HALF, 2 * EMB), jnp.float32),
            pltpu.SemaphoreType.DMA,
            pltpu.SemaphoreType.DMA,
        ],
    )
    def emb_kernel(idx_hbm, table_hbm, out_hbm, idx_v, gath0, gath1, semA, semB):
        wid = lax.axis_index("s") * NC + lax.axis_index("c")
        base = wid * b_per_w

        @pl.loop(0, macros)
        def _(i):
            off = pl.multiple_of(base + i * MACRO, MACRO)
            row0 = pl.multiple_of(off // G, MACRO // G)
            pltpu.sync_copy(idx_hbm.at[pl.ds(row0, MACRO // G)], idx_v)
            gaths = (gath0, gath1)
            sems = (semA, semB)
            nq = MACRO // HALF

            def fire(q):
                for j in range(HALF // G):
                    pltpu.async_copy(
                        table_hbm.at[idx_v.at[q * (HALF // G) + j]],
                        gaths[q % 2].at[pl.ds(j * G, G)],
                        sems[q % 2],
                    )

            def drain_out(q):
                for j in range(HALF // G):
                    pltpu.make_async_copy(
                        table_hbm.at[idx_v.at[q * (HALF // G) + j]],
                        gaths[q % 2].at[pl.ds(j * G, G)],
                        sems[q % 2],
                    ).wait()
                pltpu.sync_copy(
                    gaths[q % 2],
                    out_hbm.at[pl.ds(off + q * HALF, HALF)],
                )

            fire(0)
            for q in range(1, nq):
                fire(q)
                drain_out(q - 1)
            drain_out(nq - 1)

    out = emb_kernel(idx, table2)
    return out[:, :EMB].reshape(tokens.shape + (EMB,))


# speedup vs baseline: 2.6526x; 1.0885x over previous
import functools
import math

import jax
import jax.numpy as jnp
from jax import lax
from jax.experimental import pallas as pl
from jax.experimental.pallas import tpu as pltpu
from jax.experimental.pallas import tpu_sc as plsc

EMB = 64
SCALE = float(math.sqrt(EMB))
NC, NS = 2, 16
NW = NC * NS
G = 128
TBLK = 1024


def _transpose_scale_pad(tT):
    V = tT.shape[1]
    P = jnp.concatenate(
        [jnp.eye(EMB, dtype=jnp.bfloat16) * jnp.bfloat16(SCALE),
         jnp.zeros((EMB, EMB), jnp.bfloat16)], axis=1)

    def body(x_ref, p_ref, o_ref):
        x = x_ref[...]
        p = p_ref[...]
        xhi = x.astype(jnp.bfloat16)
        xlo = (x - xhi.astype(jnp.float32)).astype(jnp.bfloat16)
        dims = (((0,), (0,)), ((), ()))
        o_ref[...] = (
            jax.lax.dot_general(xhi, p, dims,
                                preferred_element_type=jnp.float32)
            + jax.lax.dot_general(xlo, p, dims,
                                  preferred_element_type=jnp.float32)
        )

    return pl.pallas_call(
        body,
        grid=(pl.cdiv(V, TBLK),),
        in_specs=[pl.BlockSpec((EMB, TBLK), lambda i: (0, i)),
                  pl.BlockSpec((EMB, 2 * EMB), lambda i: (0, 0))],
        out_specs=pl.BlockSpec((TBLK, 2 * EMB), lambda i: (i, 0)),
        out_shape=jax.ShapeDtypeStruct((V, 2 * EMB), jnp.float32),
        compiler_params=pltpu.CompilerParams(
            dimension_semantics=("parallel",)),
    )(tT, P)


def kernel(tokens, table):
    B = tokens.shape[0] * tokens.shape[1]
    V = table.shape[0]
    b_per_w = B // NW
    MACRO = 1024
    HALF = 256
    macros = b_per_w // MACRO
    assert b_per_w % MACRO == 0

    table2 = _transpose_scale_pad(jnp.swapaxes(table, 0, 1))
    idx = tokens.reshape(B // G, G).astype(jnp.int32)
    mesh = plsc.VectorSubcoreMesh(core_axis_name="c", subcore_axis_name="s")

    @functools.partial(
        pl.kernel,
        mesh=mesh,
        out_type=jax.ShapeDtypeStruct((B, 2 * EMB), jnp.float32),
        scratch_types=[
            pltpu.VMEM((MACRO // G, G), jnp.int32),
            pltpu.VMEM((HALF, 2 * EMB), jnp.float32),
            pltpu.VMEM((HALF, 2 * EMB), jnp.float32),
            pltpu.SemaphoreType.DMA,
            pltpu.SemaphoreType.DMA,
        ],
    )
    def emb_kernel(idx_hbm, table_hbm, out_hbm, idx_v, gath0, gath1, semA, semB):
        wid = lax.axis_index("s") * NC + lax.axis_index("c")
        base = wid * b_per_w

        @pl.loop(0, macros)
        def _(i):
            off = pl.multiple_of(base + i * MACRO, MACRO)
            row0 = pl.multiple_of(off // G, MACRO // G)
            pltpu.sync_copy(idx_hbm.at[pl.ds(row0, MACRO // G)], idx_v)
            gaths = (gath0, gath1)
            sems = (semA, semB)
            nq = MACRO // HALF

            def fire(q):
                for j in range(HALF // G):
                    pltpu.async_copy(
                        table_hbm.at[idx_v.at[q * (HALF // G) + j]],
                        gaths[q % 2].at[pl.ds(j * G, G)],
                        sems[q % 2],
                    )

            def drain_out(q):
                for j in range(HALF // G):
                    pltpu.make_async_copy(
                        table_hbm.at[idx_v.at[q * (HALF // G) + j]],
                        gaths[q % 2].at[pl.ds(j * G, G)],
                        sems[q % 2],
                    ).wait()
                pltpu.sync_copy(
                    gaths[q % 2],
                    out_hbm.at[pl.ds(off + q * HALF, HALF)],
                )

            fire(0)
            for q in range(1, nq):
                fire(q)
                drain_out(q - 1)
            drain_out(nq - 1)

    out = emb_kernel(idx, table2)
    return out[:, :EMB].reshape(tokens.shape + (EMB,))


# speedup vs baseline: 3.9308x; 1.4819x over previous
import functools
import math

import jax
import jax.numpy as jnp
from jax import lax
from jax.experimental import pallas as pl
from jax.experimental.pallas import tpu as pltpu
from jax.experimental.pallas import tpu_sc as plsc

EMB = 64
SCALE = float(math.sqrt(EMB))
NC, NS = 2, 16
NW = NC * NS
G = 128
TBLK = 4096


def _transpose_scale_pad(tT):
    V = tT.shape[1]
    P = jnp.concatenate(
        [jnp.eye(EMB, dtype=jnp.bfloat16) * jnp.bfloat16(SCALE),
         jnp.zeros((EMB, EMB), jnp.bfloat16)], axis=1)

    def body(x_ref, p_ref, o_ref):
        x = x_ref[...]
        p = p_ref[...]
        xhi = x.astype(jnp.bfloat16)
        xlo = (x - xhi.astype(jnp.float32)).astype(jnp.bfloat16)
        dims = (((0,), (0,)), ((), ()))
        o_ref[...] = (
            jax.lax.dot_general(xhi, p, dims,
                                preferred_element_type=jnp.float32)
            + jax.lax.dot_general(xlo, p, dims,
                                  preferred_element_type=jnp.float32)
        )

    return pl.pallas_call(
        body,
        grid=(pl.cdiv(V, TBLK),),
        in_specs=[pl.BlockSpec((EMB, TBLK), lambda i: (0, i)),
                  pl.BlockSpec((EMB, 2 * EMB), lambda i: (0, 0))],
        out_specs=pl.BlockSpec((TBLK, 2 * EMB), lambda i: (i, 0)),
        out_shape=jax.ShapeDtypeStruct((V, 2 * EMB), jnp.float32),
        compiler_params=pltpu.CompilerParams(
            dimension_semantics=("parallel",)),
    )(tT, P)


def kernel(tokens, table):
    B = tokens.shape[0] * tokens.shape[1]
    V = table.shape[0]
    b_per_w = B // NW
    MACRO = 1024
    HALF = 256
    macros = b_per_w // MACRO
    assert b_per_w % MACRO == 0

    table2 = _transpose_scale_pad(jnp.swapaxes(table, 0, 1))
    idx = tokens.reshape(B // G, G).astype(jnp.int32)
    mesh = plsc.VectorSubcoreMesh(core_axis_name="c", subcore_axis_name="s")

    @functools.partial(
        pl.kernel,
        mesh=mesh,
        out_type=jax.ShapeDtypeStruct((B, 2 * EMB), jnp.float32),
        scratch_types=[
            pltpu.VMEM((MACRO // G, G), jnp.int32),
            pltpu.VMEM((HALF, 2 * EMB), jnp.float32),
            pltpu.VMEM((HALF, 2 * EMB), jnp.float32),
            pltpu.SemaphoreType.DMA,
            pltpu.SemaphoreType.DMA,
        ],
    )
    def emb_kernel(idx_hbm, table_hbm, out_hbm, idx_v, gath0, gath1, semA, semB):
        wid = lax.axis_index("s") * NC + lax.axis_index("c")
        base = wid * b_per_w

        @pl.loop(0, macros)
        def _(i):
            off = pl.multiple_of(base + i * MACRO, MACRO)
            row0 = pl.multiple_of(off // G, MACRO // G)
            pltpu.sync_copy(idx_hbm.at[pl.ds(row0, MACRO // G)], idx_v)
            gaths = (gath0, gath1)
            sems = (semA, semB)
            nq = MACRO // HALF

            def fire(q):
                for j in range(HALF // G):
                    pltpu.async_copy(
                        table_hbm.at[idx_v.at[q * (HALF // G) + j]],
                        gaths[q % 2].at[pl.ds(j * G, G)],
                        sems[q % 2],
                    )

            def drain_out(q):
                for j in range(HALF // G):
                    pltpu.make_async_copy(
                        table_hbm.at[idx_v.at[q * (HALF // G) + j]],
                        gaths[q % 2].at[pl.ds(j * G, G)],
                        sems[q % 2],
                    ).wait()
                pltpu.sync_copy(
                    gaths[q % 2],
                    out_hbm.at[pl.ds(off + q * HALF, HALF)],
                )

            fire(0)
            for q in range(1, nq):
                fire(q)
                drain_out(q - 1)
            drain_out(nq - 1)

    out = emb_kernel(idx, table2)
    return out[:, :EMB].reshape(tokens.shape + (EMB,))


# speedup vs baseline: 4.4964x; 1.1439x over previous
import functools
import math

import jax
import jax.numpy as jnp
from jax import lax
from jax.experimental import pallas as pl
from jax.experimental.pallas import tpu as pltpu
from jax.experimental.pallas import tpu_sc as plsc

EMB = 64
SCALE = float(math.sqrt(EMB))
NC, NS = 2, 16
NW = NC * NS
G = 128
TBLK = 16384


def _transpose_scale_pad(tT):
    V = tT.shape[1]
    P = jnp.concatenate(
        [jnp.eye(EMB, dtype=jnp.bfloat16) * jnp.bfloat16(SCALE),
         jnp.zeros((EMB, EMB), jnp.bfloat16)], axis=1)

    def body(x_ref, p_ref, o_ref):
        x = x_ref[...]
        p = p_ref[...]
        xhi = x.astype(jnp.bfloat16)
        xlo = (x - xhi.astype(jnp.float32)).astype(jnp.bfloat16)
        dims = (((0,), (0,)), ((), ()))
        o_ref[...] = (
            jax.lax.dot_general(xhi, p, dims,
                                preferred_element_type=jnp.float32)
            + jax.lax.dot_general(xlo, p, dims,
                                  preferred_element_type=jnp.float32)
        )

    return pl.pallas_call(
        body,
        grid=(pl.cdiv(V, TBLK),),
        in_specs=[pl.BlockSpec((EMB, TBLK), lambda i: (0, i)),
                  pl.BlockSpec((EMB, 2 * EMB), lambda i: (0, 0))],
        out_specs=pl.BlockSpec((TBLK, 2 * EMB), lambda i: (i, 0)),
        out_shape=jax.ShapeDtypeStruct((V, 2 * EMB), jnp.float32),
        compiler_params=pltpu.CompilerParams(
            dimension_semantics=("parallel",)),
    )(tT, P)


def kernel(tokens, table):
    B = tokens.shape[0] * tokens.shape[1]
    V = table.shape[0]
    b_per_w = B // NW
    MACRO = 1024
    HALF = 256
    macros = b_per_w // MACRO
    assert b_per_w % MACRO == 0

    table2 = _transpose_scale_pad(jnp.swapaxes(table, 0, 1))
    idx = tokens.reshape(B // G, G).astype(jnp.int32)
    mesh = plsc.VectorSubcoreMesh(core_axis_name="c", subcore_axis_name="s")

    @functools.partial(
        pl.kernel,
        mesh=mesh,
        out_type=jax.ShapeDtypeStruct((B, 2 * EMB), jnp.float32),
        scratch_types=[
            pltpu.VMEM((MACRO // G, G), jnp.int32),
            pltpu.VMEM((HALF, 2 * EMB), jnp.float32),
            pltpu.VMEM((HALF, 2 * EMB), jnp.float32),
            pltpu.SemaphoreType.DMA,
            pltpu.SemaphoreType.DMA,
        ],
    )
    def emb_kernel(idx_hbm, table_hbm, out_hbm, idx_v, gath0, gath1, semA, semB):
        wid = lax.axis_index("s") * NC + lax.axis_index("c")
        base = wid * b_per_w

        @pl.loop(0, macros)
        def _(i):
            off = pl.multiple_of(base + i * MACRO, MACRO)
            row0 = pl.multiple_of(off // G, MACRO // G)
            pltpu.sync_copy(idx_hbm.at[pl.ds(row0, MACRO // G)], idx_v)
            gaths = (gath0, gath1)
            sems = (semA, semB)
            nq = MACRO // HALF

            def fire(q):
                for j in range(HALF // G):
                    pltpu.async_copy(
                        table_hbm.at[idx_v.at[q * (HALF // G) + j]],
                        gaths[q % 2].at[pl.ds(j * G, G)],
                        sems[q % 2],
                    )

            def drain_out(q):
                for j in range(HALF // G):
                    pltpu.make_async_copy(
                        table_hbm.at[idx_v.at[q * (HALF // G) + j]],
                        gaths[q % 2].at[pl.ds(j * G, G)],
                        sems[q % 2],
                    ).wait()
                pltpu.sync_copy(
                    gaths[q % 2],
                    out_hbm.at[pl.ds(off + q * HALF, HALF)],
                )

            fire(0)
            for q in range(1, nq):
                fire(q)
                drain_out(q - 1)
            drain_out(nq - 1)

    out = emb_kernel(idx, table2)
    return out[:, :EMB].reshape(tokens.shape + (EMB,))


# speedup vs baseline: 4.5333x; 1.0082x over previous
import functools
import math

import jax
import jax.numpy as jnp
from jax import lax
from jax.experimental import pallas as pl
from jax.experimental.pallas import tpu as pltpu
from jax.experimental.pallas import tpu_sc as plsc

EMB = 64
SCALE = float(math.sqrt(EMB))
NC, NS = 2, 16
NW = NC * NS
G = 128
TBLK = 32768


def _transpose_scale_pad(tT):
    V = tT.shape[1]
    P = jnp.concatenate(
        [jnp.eye(EMB, dtype=jnp.bfloat16) * jnp.bfloat16(SCALE),
         jnp.zeros((EMB, EMB), jnp.bfloat16)], axis=1)

    def body(x_ref, p_ref, o_ref):
        x = x_ref[...]
        p = p_ref[...]
        xhi = x.astype(jnp.bfloat16)
        xlo = (x - xhi.astype(jnp.float32)).astype(jnp.bfloat16)
        dims = (((0,), (0,)), ((), ()))
        o_ref[...] = (
            jax.lax.dot_general(xhi, p, dims,
                                preferred_element_type=jnp.float32)
            + jax.lax.dot_general(xlo, p, dims,
                                  preferred_element_type=jnp.float32)
        )

    return pl.pallas_call(
        body,
        grid=(pl.cdiv(V, TBLK),),
        in_specs=[pl.BlockSpec((EMB, TBLK), lambda i: (0, i)),
                  pl.BlockSpec((EMB, 2 * EMB), lambda i: (0, 0))],
        out_specs=pl.BlockSpec((TBLK, 2 * EMB), lambda i: (i, 0)),
        out_shape=jax.ShapeDtypeStruct((V, 2 * EMB), jnp.float32),
        compiler_params=pltpu.CompilerParams(
            dimension_semantics=("parallel",)),
    )(tT, P)


def kernel(tokens, table):
    B = tokens.shape[0] * tokens.shape[1]
    V = table.shape[0]
    b_per_w = B // NW
    MACRO = 1024
    HALF = 256
    macros = b_per_w // MACRO
    assert b_per_w % MACRO == 0

    table2 = _transpose_scale_pad(jnp.swapaxes(table, 0, 1))
    idx = tokens.reshape(B // G, G).astype(jnp.int32)
    mesh = plsc.VectorSubcoreMesh(core_axis_name="c", subcore_axis_name="s")

    @functools.partial(
        pl.kernel,
        mesh=mesh,
        out_type=jax.ShapeDtypeStruct((B, 2 * EMB), jnp.float32),
        scratch_types=[
            pltpu.VMEM((MACRO // G, G), jnp.int32),
            pltpu.VMEM((HALF, 2 * EMB), jnp.float32),
            pltpu.VMEM((HALF, 2 * EMB), jnp.float32),
            pltpu.SemaphoreType.DMA,
            pltpu.SemaphoreType.DMA,
        ],
    )
    def emb_kernel(idx_hbm, table_hbm, out_hbm, idx_v, gath0, gath1, semA, semB):
        wid = lax.axis_index("s") * NC + lax.axis_index("c")
        base = wid * b_per_w

        @pl.loop(0, macros)
        def _(i):
            off = pl.multiple_of(base + i * MACRO, MACRO)
            row0 = pl.multiple_of(off // G, MACRO // G)
            pltpu.sync_copy(idx_hbm.at[pl.ds(row0, MACRO // G)], idx_v)
            gaths = (gath0, gath1)
            sems = (semA, semB)
            nq = MACRO // HALF

            def fire(q):
                for j in range(HALF // G):
                    pltpu.async_copy(
                        table_hbm.at[idx_v.at[q * (HALF // G) + j]],
                        gaths[q % 2].at[pl.ds(j * G, G)],
                        sems[q % 2],
                    )

            def drain_out(q):
                for j in range(HALF // G):
                    pltpu.make_async_copy(
                        table_hbm.at[idx_v.at[q * (HALF // G) + j]],
                        gaths[q % 2].at[pl.ds(j * G, G)],
                        sems[q % 2],
                    ).wait()
                pltpu.sync_copy(
                    gaths[q % 2],
                    out_hbm.at[pl.ds(off + q * HALF, HALF)],
                )

            fire(0)
            for q in range(1, nq):
                fire(q)
                drain_out(q - 1)
            drain_out(nq - 1)

    out = emb_kernel(idx, table2)
    return out[:, :EMB].reshape(tokens.shape + (EMB,))
